# baseline (speedup 1.0000x reference)
import functools

import jax
import jax.numpy as jnp
from jax import lax
from jax.experimental import pallas as pl
from jax.experimental.pallas import tpu as pltpu
from jax.experimental.pallas import tpu_sc as plsc

S = 2048
D = 768
E = 8
DFF = 3072
BLK = 128
NBLK = (2 * S) // BLK + E
NPAD = NBLK * BLK
CHUNK = 128


def _route_body(x_ref, wg_ref, xg0_ref, xg1_ref, pos0_ref, pos1_ref,
                bids_ref, used_ref, ff_ref, ordb_ref, uniq_ref, cex_ref):
    xf = x_ref[...]
    logits = lax.dot_general(xf, wg_ref[...], (((1,), (1,)), ((), ())),
                             preferred_element_type=jnp.float32)
    eidx = lax.broadcasted_iota(jnp.int32, (S, E), 1)
    m1 = jnp.max(logits, axis=1, keepdims=True)
    i1 = jnp.min(jnp.where(logits == m1, eidx, E), axis=1, keepdims=True)
    masked = jnp.where(eidx == i1, -jnp.inf, logits)
    m2 = jnp.max(masked, axis=1, keepdims=True)
    i2 = jnp.min(jnp.where(masked == m2, eidx, E), axis=1, keepdims=True)
    g1 = 1.0 / (1.0 + jnp.exp(m2 - m1))
    g2 = 1.0 - g1
    a1 = eidx == i1
    a2 = eidx == i2
    m = jnp.where(a1, 1.0, 0.0) + jnp.where(a2, 1.0, 0.0)

    r = lax.broadcasted_iota(jnp.int32, (CHUNK, CHUNK), 0)
    c = lax.broadcasted_iota(jnp.int32, (CHUNK, CHUNK), 1)
    ltri = jnp.where(r > c, 1.0, 0.0)
    carry = jnp.zeros((1, E), jnp.float32)
    for k in range(S // CHUNK):
        mc = m[k * CHUNK:(k + 1) * CHUNK, :]
        cex_ref[k * CHUNK:(k + 1) * CHUNK, :] = carry + lax.dot_general(
            ltri, mc, (((1,), (0,)), ((), ())),
            preferred_element_type=jnp.float32)
        carry = carry + jnp.sum(mc, axis=0, keepdims=True)

    counts = carry
    padded = jnp.ceil(counts / BLK) * BLK
    uidx_r = lax.broadcasted_iota(jnp.int32, (E, E), 0)
    uidx_c = lax.broadcasted_iota(jnp.int32, (E, E), 1)
    utri = jnp.where(uidx_r < uidx_c, 1.0, 0.0)
    start = lax.dot_general(padded, utri, (((1,), (0,)), ((), ())),
                            preferred_element_type=jnp.float32)
    base = start + cex_ref[...]
    pos0 = jnp.sum(jnp.where(a1, base, 0.0), axis=1, keepdims=True)
    pos1 = jnp.sum(jnp.where(a2, base, 0.0), axis=1, keepdims=True)
    pos0_ref[...] = pos0.astype(jnp.int32)
    pos1_ref[...] = pos1.astype(jnp.int32)
    xg0_ref[...] = g1 * xf
    xg1_ref[...] = g2 * xf

    nblk_f = padded / BLK
    cuminc = lax.dot_general(nblk_f, jnp.where(uidx_r <= uidx_c, 1.0, 0.0),
                             (((1,), (0,)), ((), ())),
                             preferred_element_type=jnp.float32)
    jj = lax.broadcasted_iota(jnp.int32, (NBLK, E), 0).astype(jnp.float32)
    bid_raw = jnp.sum(jnp.where(jj >= cuminc, 1.0, 0.0), axis=1,
                      keepdims=True)
    total = jnp.max(cuminc, axis=1, keepdims=True)
    jcol = lax.broadcasted_iota(jnp.int32, (NBLK, 1), 0).astype(jnp.float32)
    used = jnp.where(jcol < total, 1.0, 0.0)
    erow = lax.broadcasted_iota(jnp.int32, (1, E), 1).astype(jnp.float32)
    last_bid = jnp.max(jnp.where(nblk_f > 0.0, erow, 0.0), axis=1,
                       keepdims=True)
    bids = jnp.where(used > 0.0, jnp.minimum(bid_raw, float(E - 1)),
                     last_bid)
    bids_ref[...] = bids.astype(jnp.int32)
    used_ref[...] = used.astype(jnp.int32)

    ind = jnp.where(nblk_f > 0.0, 1.0, 0.0)
    rk = lax.dot_general(ind, jnp.where(uidx_r < uidx_c, 1.0, 0.0),
                         (((1,), (0,)), ((), ())),
                         preferred_element_type=jnp.float32)
    rr = lax.broadcasted_iota(jnp.int32, (E, E), 0).astype(jnp.float32)
    uq_mat = jnp.where((rk == rr) & (ind > 0.0), erow, 0.0)
    uniq = jnp.sum(uq_mat, axis=1, keepdims=True)
    nuniq = jnp.sum(ind, axis=1, keepdims=True)
    uniq_ref[...] = jnp.concatenate([uniq, nuniq], axis=0).astype(jnp.int32)
    eq = bids == erow
    ordb = jnp.sum(jnp.where(eq, rk, 0.0), axis=1, keepdims=True)
    ordb_ref[...] = ordb.astype(jnp.int32)
    prev = jnp.concatenate([-jnp.ones((1, 1), jnp.float32), bids[:-1]], axis=0)
    ff = jnp.where((used > 0.0) & (bids != prev), 1.0, 0.0)
    ff_ref[...] = ff.astype(jnp.int32)


def _route(x2, wg):
    return pl.pallas_call(
        _route_body,
        out_shape=(
            jax.ShapeDtypeStruct((S, D), jnp.float32),
            jax.ShapeDtypeStruct((S, D), jnp.float32),
            jax.ShapeDtypeStruct((S, 1), jnp.int32),
            jax.ShapeDtypeStruct((S, 1), jnp.int32),
            jax.ShapeDtypeStruct((NBLK, 1), jnp.int32),
            jax.ShapeDtypeStruct((NBLK, 1), jnp.int32),
            jax.ShapeDtypeStruct((NBLK, 1), jnp.int32),
            jax.ShapeDtypeStruct((NBLK, 1), jnp.int32),
            jax.ShapeDtypeStruct((E + 1, 1), jnp.int32),
        ),
        scratch_shapes=[pltpu.VMEM((S, E), jnp.float32)],
    )(x2, wg)


def _make_scatter():
    info = plsc.get_sparse_core_info()
    nc, ns = info.num_cores, info.num_subcores
    nw = nc * ns
    rw = S // nw
    mesh = plsc.VectorSubcoreMesh(core_axis_name="c", subcore_axis_name="s")

    @functools.partial(
        pl.kernel, mesh=mesh,
        out_type=jax.ShapeDtypeStruct((NPAD, D), jnp.float32),
        scratch_types=[
            pltpu.VMEM((rw,), jnp.int32),
            pltpu.VMEM((rw,), jnp.int32),
            pltpu.VMEM((rw, D), jnp.float32),
            pltpu.VMEM((rw, D), jnp.float32),
            pltpu.SemaphoreType.DMA,
            pltpu.SemaphoreType.DMA,
        ],
    )
    def scatter_k(xg0_hbm, xg1_hbm, pos0_hbm, pos1_hbm, xs_hbm,
                  idx0_v, idx1_v, rows0_v, rows1_v, sem0, sem1):
        wid = lax.axis_index("s") * nc + lax.axis_index("c")
        b = wid * rw
        pltpu.sync_copy(pos0_hbm.at[pl.ds(b, rw)], idx0_v)
        pltpu.sync_copy(pos1_hbm.at[pl.ds(b, rw)], idx1_v)
        pltpu.sync_copy(xg0_hbm.at[pl.ds(b, rw)], rows0_v)
        pltpu.sync_copy(xg1_hbm.at[pl.ds(b, rw)], rows1_v)
        c0 = pltpu.async_copy(rows0_v, xs_hbm.at[idx0_v], sem0)
        c1 = pltpu.async_copy(rows1_v, xs_hbm.at[idx1_v], sem1)
        c0.wait()
        c1.wait()

    return scatter_k


def _ffn_body(bids_ref, used_ref, ff_ref, ordb_ref, uniq_ref,
              xs_ref, w1_hbm, w2_hbm, ys_ref, w1b, w2b, s1, s2):
    b = pl.program_id(0)
    o = ordb_ref[b]
    nuniq = uniq_ref[E]

    @pl.when(b == 0)
    def _():
        pltpu.make_async_copy(w1_hbm.at[uniq_ref[0]], w1b.at[0],
                              s1.at[0]).start()
        pltpu.make_async_copy(w2_hbm.at[uniq_ref[0]], w2b.at[0],
                              s2.at[0]).start()

        @pl.when(nuniq > 1)
        def _():
            pltpu.make_async_copy(w1_hbm.at[uniq_ref[1]], w1b.at[1],
                                  s1.at[1]).start()
            pltpu.make_async_copy(w2_hbm.at[uniq_ref[1]], w2b.at[1],
                                  s2.at[1]).start()

        @pl.when(nuniq > 2)
        def _():
            pltpu.make_async_copy(w1_hbm.at[uniq_ref[2]], w1b.at[2],
                                  s1.at[2]).start()
            pltpu.make_async_copy(w2_hbm.at[uniq_ref[2]], w2b.at[2],
                                  s2.at[2]).start()

    first = ff_ref[b] == 1
    o3 = lax.rem(o, 3)

    @pl.when(first & (b > 0) & (o + 2 < nuniq))
    def _():
        nxt = uniq_ref[o + 2]
        n3 = lax.rem(o + 2, 3)

        @pl.when(n3 == 0)
        def _():
            pltpu.make_async_copy(w1_hbm.at[nxt], w1b.at[0], s1.at[0]).start()
            pltpu.make_async_copy(w2_hbm.at[nxt], w2b.at[0], s2.at[0]).start()

        @pl.when(n3 == 1)
        def _():
            pltpu.make_async_copy(w1_hbm.at[nxt], w1b.at[1], s1.at[1]).start()
            pltpu.make_async_copy(w2_hbm.at[nxt], w2b.at[1], s2.at[1]).start()

        @pl.when(n3 == 2)
        def _():
            pltpu.make_async_copy(w1_hbm.at[nxt], w1b.at[2], s1.at[2]).start()
            pltpu.make_async_copy(w2_hbm.at[nxt], w2b.at[2], s2.at[2]).start()

    @pl.when(first)
    def _():
        @pl.when(o3 == 0)
        def _():
            pltpu.make_async_copy(w1_hbm.at[0], w1b.at[0], s1.at[0]).wait()
            pltpu.make_async_copy(w2_hbm.at[0], w2b.at[0], s2.at[0]).wait()

        @pl.when(o3 == 1)
        def _():
            pltpu.make_async_copy(w1_hbm.at[0], w1b.at[1], s1.at[1]).wait()
            pltpu.make_async_copy(w2_hbm.at[0], w2b.at[1], s2.at[1]).wait()

        @pl.when(o3 == 2)
        def _():
            pltpu.make_async_copy(w1_hbm.at[0], w1b.at[2], s1.at[2]).wait()
            pltpu.make_async_copy(w2_hbm.at[0], w2b.at[2], s2.at[2]).wait()

    @pl.when(used_ref[b] == 1)
    def _():
        xb = xs_ref[...]
        hdff = DFF // 2
        acc = jnp.zeros((BLK, D), jnp.float32)
        for ch in range(2):
            w1v = w1b[o3, ch * hdff:(ch + 1) * hdff, :]
            w2v = w2b[o3, :, ch * hdff:(ch + 1) * hdff]
            h = lax.dot_general(xb, w1v, (((1,), (1,)), ((), ())),
                                preferred_element_type=jnp.float32)
            h = jnp.maximum(h, 0.0)
            acc = acc + lax.dot_general(h, w2v, (((1,), (1,)), ((), ())),
                                        preferred_element_type=jnp.float32)
        ys_ref[...] = acc


def _ffn(bids, used, ff, ordb, uniq, xs, w1, w2):
    grid_spec = pltpu.PrefetchScalarGridSpec(
        num_scalar_prefetch=5,
        grid=(NBLK,),
        in_specs=[
            pl.BlockSpec((BLK, D), lambda b, *_: (b, 0)),
            pl.BlockSpec(memory_space=pl.ANY),
            pl.BlockSpec(memory_space=pl.ANY),
        ],
        out_specs=pl.BlockSpec((BLK, D), lambda b, *_: (b, 0)),
        scratch_shapes=[
            pltpu.VMEM((3, DFF, D), jnp.float32),
            pltpu.VMEM((3, D, DFF), jnp.float32),
            pltpu.SemaphoreType.DMA((3,)),
            pltpu.SemaphoreType.DMA((3,)),
        ],
    )
    return pl.pallas_call(
        _ffn_body,
        grid_spec=grid_spec,
        out_shape=jax.ShapeDtypeStruct((NPAD, D), jnp.float32),
        compiler_params=pltpu.CompilerParams(
            dimension_semantics=("arbitrary",)),
    )(bids, used, ff, ordb, uniq, xs, w1, w2)


def _make_combine():
    info = plsc.get_sparse_core_info()
    nc, ns = info.num_cores, info.num_subcores
    nw = nc * ns
    rw = S // nw
    nv = D // 16
    mesh = plsc.VectorSubcoreMesh(core_axis_name="c", subcore_axis_name="s")

    @functools.partial(
        pl.kernel, mesh=mesh,
        out_type=jax.ShapeDtypeStruct((S, D), jnp.float32),
        scratch_types=[
            pltpu.VMEM((rw,), jnp.int32),
            pltpu.VMEM((rw,), jnp.int32),
            pltpu.VMEM((rw, D), jnp.float32),
            pltpu.VMEM((rw, D), jnp.float32),
            pltpu.SemaphoreType.DMA,
            pltpu.SemaphoreType.DMA,
        ],
    )
    def combine_k(ys_hbm, pos0_hbm, pos1_hbm, out_hbm,
                  idx0_v, idx1_v, bufa_v, bufb_v, sem0, sem1):
        wid = lax.axis_index("s") * nc + lax.axis_index("c")
        b = wid * rw
        pltpu.sync_copy(pos0_hbm.at[pl.ds(b, rw)], idx0_v)
        pltpu.sync_copy(pos1_hbm.at[pl.ds(b, rw)], idx1_v)
        ca = pltpu.async_copy(ys_hbm.at[idx0_v], bufa_v, sem0)
        cb = pltpu.async_copy(ys_hbm.at[idx1_v], bufb_v, sem1)
        ca.wait()
        cb.wait()

        def row(r, _):
            for cc in range(nv):
                sl = pl.ds(cc * 16, 16)
                bufa_v[r, sl] = bufa_v[r, sl] + bufb_v[r, sl]
            return 0

        lax.fori_loop(0, rw, row, 0)
        pltpu.sync_copy(bufa_v, out_hbm.at[pl.ds(b, rw)])

    return combine_k


def kernel(x, Wg, W1, W2):
    bsz, s, d = x.shape
    x2 = x.reshape(s, d)

    xg0, xg1, pos0, pos1, bids, used, ff, ordb, uniq = _route(x2, Wg)
    pos0 = pos0.reshape(S)
    pos1 = pos1.reshape(S)
    bids = bids.reshape(NBLK)
    used = used.reshape(NBLK)
    ff = ff.reshape(NBLK)
    ordb = ordb.reshape(NBLK)
    uniq = uniq.reshape(E + 1)

    xs = _make_scatter()(xg0, xg1, pos0, pos1)
    ys = _ffn(bids, used, ff, ordb, uniq, xs, W1, W2)
    out = _make_combine()(ys, pos0, pos1)
    return out.reshape(bsz, s, d)


# speedup vs baseline: 1.4574x; 1.4574x over previous
import functools

import jax
import jax.numpy as jnp
from jax import lax
from jax.experimental import pallas as pl
from jax.experimental.pallas import tpu as pltpu
from jax.experimental.pallas import tpu_sc as plsc

S = 2048
D = 768
E = 8
DFF = 3072
BLK = 256
NBLK = (2 * S) // BLK + E
NPAD = NBLK * BLK
CHUNK = 128


def _route_body(x_ref, wg_ref, xg0_ref, xg1_ref, pos0_ref, pos1_ref,
                bids_ref, used_ref, ff_ref, ordb_ref, uniq_ref, cex_ref):
    xf = x_ref[...]
    logits = lax.dot_general(xf, wg_ref[...], (((1,), (1,)), ((), ())),
                             preferred_element_type=jnp.float32)
    eidx = lax.broadcasted_iota(jnp.int32, (S, E), 1)
    m1 = jnp.max(logits, axis=1, keepdims=True)
    i1 = jnp.min(jnp.where(logits == m1, eidx, E), axis=1, keepdims=True)
    masked = jnp.where(eidx == i1, -jnp.inf, logits)
    m2 = jnp.max(masked, axis=1, keepdims=True)
    i2 = jnp.min(jnp.where(masked == m2, eidx, E), axis=1, keepdims=True)
    g1 = 1.0 / (1.0 + jnp.exp(m2 - m1))
    g2 = 1.0 - g1
    a1 = eidx == i1
    a2 = eidx == i2
    m = jnp.where(a1, 1.0, 0.0) + jnp.where(a2, 1.0, 0.0)

    r = lax.broadcasted_iota(jnp.int32, (CHUNK, CHUNK), 0)
    c = lax.broadcasted_iota(jnp.int32, (CHUNK, CHUNK), 1)
    ltri = jnp.where(r > c, 1.0, 0.0)
    carry = jnp.zeros((1, E), jnp.float32)
    for k in range(S // CHUNK):
        mc = m[k * CHUNK:(k + 1) * CHUNK, :]
        cex_ref[k * CHUNK:(k + 1) * CHUNK, :] = carry + lax.dot_general(
            ltri, mc, (((1,), (0,)), ((), ())),
            preferred_element_type=jnp.float32)
        carry = carry + jnp.sum(mc, axis=0, keepdims=True)

    counts = carry
    padded = jnp.ceil(counts / BLK) * BLK
    uidx_r = lax.broadcasted_iota(jnp.int32, (E, E), 0)
    uidx_c = lax.broadcasted_iota(jnp.int32, (E, E), 1)
    utri = jnp.where(uidx_r < uidx_c, 1.0, 0.0)
    start = lax.dot_general(padded, utri, (((1,), (0,)), ((), ())),
                            preferred_element_type=jnp.float32)
    base = start + cex_ref[...]
    pos0 = jnp.sum(jnp.where(a1, base, 0.0), axis=1, keepdims=True)
    pos1 = jnp.sum(jnp.where(a2, base, 0.0), axis=1, keepdims=True)
    pos0_ref[...] = pos0.astype(jnp.int32)
    pos1_ref[...] = pos1.astype(jnp.int32)
    xg0_ref[...] = g1 * xf
    xg1_ref[...] = g2 * xf

    nblk_f = padded / BLK
    cuminc = lax.dot_general(nblk_f, jnp.where(uidx_r <= uidx_c, 1.0, 0.0),
                             (((1,), (0,)), ((), ())),
                             preferred_element_type=jnp.float32)
    jj = lax.broadcasted_iota(jnp.int32, (NBLK, E), 0).astype(jnp.float32)
    bid_raw = jnp.sum(jnp.where(jj >= cuminc, 1.0, 0.0), axis=1,
                      keepdims=True)
    total = jnp.max(cuminc, axis=1, keepdims=True)
    jcol = lax.broadcasted_iota(jnp.int32, (NBLK, 1), 0).astype(jnp.float32)
    used = jnp.where(jcol < total, 1.0, 0.0)
    erow = lax.broadcasted_iota(jnp.int32, (1, E), 1).astype(jnp.float32)
    last_bid = jnp.max(jnp.where(nblk_f > 0.0, erow, 0.0), axis=1,
                       keepdims=True)
    bids = jnp.where(used > 0.0, jnp.minimum(bid_raw, float(E - 1)),
                     last_bid)
    bids_ref[...] = bids.astype(jnp.int32)
    used_ref[...] = used.astype(jnp.int32)

    ind = jnp.where(nblk_f > 0.0, 1.0, 0.0)
    rk = lax.dot_general(ind, jnp.where(uidx_r < uidx_c, 1.0, 0.0),
                         (((1,), (0,)), ((), ())),
                         preferred_element_type=jnp.float32)
    rr = lax.broadcasted_iota(jnp.int32, (E, E), 0).astype(jnp.float32)
    uq_mat = jnp.where((rk == rr) & (ind > 0.0), erow, 0.0)
    uniq = jnp.sum(uq_mat, axis=1, keepdims=True)
    nuniq = jnp.sum(ind, axis=1, keepdims=True)
    uniq_ref[...] = jnp.concatenate([uniq, nuniq], axis=0).astype(jnp.int32)
    eq = bids == erow
    ordb = jnp.sum(jnp.where(eq, rk, 0.0), axis=1, keepdims=True)
    ordb_ref[...] = ordb.astype(jnp.int32)
    prev = jnp.concatenate([-jnp.ones((1, 1), jnp.float32), bids[:-1]], axis=0)
    ff = jnp.where((used > 0.0) & (bids != prev), 1.0, 0.0)
    ff_ref[...] = ff.astype(jnp.int32)


def _route(x2, wg):
    return pl.pallas_call(
        _route_body,
        out_shape=(
            jax.ShapeDtypeStruct((S, D), jnp.float32),
            jax.ShapeDtypeStruct((S, D), jnp.float32),
            jax.ShapeDtypeStruct((S, 1), jnp.int32),
            jax.ShapeDtypeStruct((S, 1), jnp.int32),
            jax.ShapeDtypeStruct((NBLK, 1), jnp.int32),
            jax.ShapeDtypeStruct((NBLK, 1), jnp.int32),
            jax.ShapeDtypeStruct((NBLK, 1), jnp.int32),
            jax.ShapeDtypeStruct((NBLK, 1), jnp.int32),
            jax.ShapeDtypeStruct((E + 1, 1), jnp.int32),
        ),
        scratch_shapes=[pltpu.VMEM((S, E), jnp.float32)],
    )(x2, wg)


def _make_scatter():
    info = plsc.get_sparse_core_info()
    nc, ns = info.num_cores, info.num_subcores
    nw = nc * ns
    rw = S // nw
    mesh = plsc.VectorSubcoreMesh(core_axis_name="c", subcore_axis_name="s")

    @functools.partial(
        pl.kernel, mesh=mesh,
        out_type=jax.ShapeDtypeStruct((NPAD, D), jnp.float32),
        scratch_types=[
            pltpu.VMEM((rw,), jnp.int32),
            pltpu.VMEM((rw,), jnp.int32),
            pltpu.VMEM((rw, D), jnp.float32),
            pltpu.VMEM((rw, D), jnp.float32),
            pltpu.SemaphoreType.DMA,
            pltpu.SemaphoreType.DMA,
        ],
    )
    def scatter_k(xg0_hbm, xg1_hbm, pos0_hbm, pos1_hbm, xs_hbm,
                  idx0_v, idx1_v, rows0_v, rows1_v, sem0, sem1):
        wid = lax.axis_index("s") * nc + lax.axis_index("c")
        b = wid * rw
        pltpu.sync_copy(pos0_hbm.at[pl.ds(b, rw)], idx0_v)
        pltpu.sync_copy(pos1_hbm.at[pl.ds(b, rw)], idx1_v)
        pltpu.sync_copy(xg0_hbm.at[pl.ds(b, rw)], rows0_v)
        pltpu.sync_copy(xg1_hbm.at[pl.ds(b, rw)], rows1_v)
        c0 = pltpu.async_copy(rows0_v, xs_hbm.at[idx0_v], sem0)
        c1 = pltpu.async_copy(rows1_v, xs_hbm.at[idx1_v], sem1)
        c0.wait()
        c1.wait()

    return scatter_k


def _ffn_body(bids_ref, used_ref, ff_ref, ordb_ref, uniq_ref,
              xs_ref, w1_hbm, w2_hbm, ys_ref, w1b, w2b, s1, s2):
    b = pl.program_id(0)
    o = ordb_ref[b]
    nuniq = uniq_ref[E]

    @pl.when(b == 0)
    def _():
        pltpu.make_async_copy(w1_hbm.at[uniq_ref[0]], w1b.at[0],
                              s1.at[0]).start()
        pltpu.make_async_copy(w2_hbm.at[uniq_ref[0]], w2b.at[0],
                              s2.at[0]).start()

        @pl.when(nuniq > 1)
        def _():
            pltpu.make_async_copy(w1_hbm.at[uniq_ref[1]], w1b.at[1],
                                  s1.at[1]).start()
            pltpu.make_async_copy(w2_hbm.at[uniq_ref[1]], w2b.at[1],
                                  s2.at[1]).start()

        @pl.when(nuniq > 2)
        def _():
            pltpu.make_async_copy(w1_hbm.at[uniq_ref[2]], w1b.at[2],
                                  s1.at[2]).start()

    first = ff_ref[b] == 1
    o3 = lax.rem(o, 3)
    o2 = lax.rem(o, 2)

    @pl.when(first & (b > 0) & (o + 2 < nuniq))
    def _():
        nxt = uniq_ref[o + 2]
        n3 = lax.rem(o + 2, 3)

        @pl.when(n3 == 0)
        def _():
            pltpu.make_async_copy(w1_hbm.at[nxt], w1b.at[0], s1.at[0]).start()

        @pl.when(n3 == 1)
        def _():
            pltpu.make_async_copy(w1_hbm.at[nxt], w1b.at[1], s1.at[1]).start()

        @pl.when(n3 == 2)
        def _():
            pltpu.make_async_copy(w1_hbm.at[nxt], w1b.at[2], s1.at[2]).start()

    @pl.when(first & (b > 0) & (o + 1 < nuniq))
    def _():
        nx1 = uniq_ref[o + 1]
        m2 = lax.rem(o + 1, 2)

        @pl.when(m2 == 0)
        def _():
            pltpu.make_async_copy(w2_hbm.at[nx1], w2b.at[0], s2.at[0]).start()

        @pl.when(m2 == 1)
        def _():
            pltpu.make_async_copy(w2_hbm.at[nx1], w2b.at[1], s2.at[1]).start()

    @pl.when(first)
    def _():
        @pl.when(o3 == 0)
        def _():
            pltpu.make_async_copy(w1_hbm.at[0], w1b.at[0], s1.at[0]).wait()

        @pl.when(o3 == 1)
        def _():
            pltpu.make_async_copy(w1_hbm.at[0], w1b.at[1], s1.at[1]).wait()

        @pl.when(o3 == 2)
        def _():
            pltpu.make_async_copy(w1_hbm.at[0], w1b.at[2], s1.at[2]).wait()

        @pl.when(o2 == 0)
        def _():
            pltpu.make_async_copy(w2_hbm.at[0], w2b.at[0], s2.at[0]).wait()

        @pl.when(o2 == 1)
        def _():
            pltpu.make_async_copy(w2_hbm.at[0], w2b.at[1], s2.at[1]).wait()

    @pl.when(used_ref[b] == 1)
    def _():
        xb = xs_ref[...]
        w1v = w1b[o3]
        w2v = w2b[o2]
        h = lax.dot_general(xb, w1v, (((1,), (1,)), ((), ())),
                            preferred_element_type=jnp.float32)
        h = jnp.maximum(h, 0.0)
        ys_ref[...] = lax.dot_general(h, w2v, (((1,), (1,)), ((), ())),
                                      preferred_element_type=jnp.float32)


def _ffn(bids, used, ff, ordb, uniq, xs, w1, w2):
    grid_spec = pltpu.PrefetchScalarGridSpec(
        num_scalar_prefetch=5,
        grid=(NBLK,),
        in_specs=[
            pl.BlockSpec((BLK, D), lambda b, *_: (b, 0)),
            pl.BlockSpec(memory_space=pl.ANY),
            pl.BlockSpec(memory_space=pl.ANY),
        ],
        out_specs=pl.BlockSpec((BLK, D), lambda b, *_: (b, 0)),
        scratch_shapes=[
            pltpu.VMEM((3, DFF, D), jnp.float32),
            pltpu.VMEM((2, D, DFF), jnp.float32),
            pltpu.SemaphoreType.DMA((3,)),
            pltpu.SemaphoreType.DMA((2,)),
        ],
    )
    return pl.pallas_call(
        _ffn_body,
        grid_spec=grid_spec,
        out_shape=jax.ShapeDtypeStruct((NPAD, D), jnp.float32),
        compiler_params=pltpu.CompilerParams(
            dimension_semantics=("arbitrary",)),
    )(bids, used, ff, ordb, uniq, xs, w1, w2)


def _make_combine():
    info = plsc.get_sparse_core_info()
    nc, ns = info.num_cores, info.num_subcores
    nw = nc * ns
    rw = S // nw
    nv = D // 16
    mesh = plsc.VectorSubcoreMesh(core_axis_name="c", subcore_axis_name="s")

    @functools.partial(
        pl.kernel, mesh=mesh,
        out_type=jax.ShapeDtypeStruct((S, D), jnp.float32),
        scratch_types=[
            pltpu.VMEM((rw,), jnp.int32),
            pltpu.VMEM((rw,), jnp.int32),
            pltpu.VMEM((rw, D), jnp.float32),
            pltpu.VMEM((rw, D), jnp.float32),
            pltpu.SemaphoreType.DMA,
            pltpu.SemaphoreType.DMA,
        ],
    )
    def combine_k(ys_hbm, pos0_hbm, pos1_hbm, out_hbm,
                  idx0_v, idx1_v, bufa_v, bufb_v, sem0, sem1):
        wid = lax.axis_index("s") * nc + lax.axis_index("c")
        b = wid * rw
        pltpu.sync_copy(pos0_hbm.at[pl.ds(b, rw)], idx0_v)
        pltpu.sync_copy(pos1_hbm.at[pl.ds(b, rw)], idx1_v)
        ca = pltpu.async_copy(ys_hbm.at[idx0_v], bufa_v, sem0)
        cb = pltpu.async_copy(ys_hbm.at[idx1_v], bufb_v, sem1)
        ca.wait()
        cb.wait()

        def row(r, _):
            for cc in range(nv):
                sl = pl.ds(cc * 16, 16)
                bufa_v[r, sl] = bufa_v[r, sl] + bufb_v[r, sl]
            return 0

        lax.fori_loop(0, rw, row, 0)
        pltpu.sync_copy(bufa_v, out_hbm.at[pl.ds(b, rw)])

    return combine_k


def kernel(x, Wg, W1, W2):
    bsz, s, d = x.shape
    x2 = x.reshape(s, d)

    xg0, xg1, pos0, pos1, bids, used, ff, ordb, uniq = _route(x2, Wg)
    pos0 = pos0.reshape(S)
    pos1 = pos1.reshape(S)
    bids = bids.reshape(NBLK)
    used = used.reshape(NBLK)
    ff = ff.reshape(NBLK)
    ordb = ordb.reshape(NBLK)
    uniq = uniq.reshape(E + 1)

    xs = _make_scatter()(xg0, xg1, pos0, pos1)
    ys = _ffn(bids, used, ff, ordb, uniq, xs, W1, W2)
    out = _make_combine()(ys, pos0, pos1)
    return out.reshape(bsz, s, d)


# speedup vs baseline: 1.4776x; 1.0138x over previous
import functools

import jax
import jax.numpy as jnp
from jax import lax
from jax.experimental import pallas as pl
from jax.experimental.pallas import tpu as pltpu
from jax.experimental.pallas import tpu_sc as plsc

S = 2048
D = 768
E = 8
DFF = 3072
BLK = 256
NBLK = (2 * S) // BLK + E
NPAD = NBLK * BLK
CHUNK = 128


def _route_body(x_ref, wg_ref, xg0_ref, xg1_ref, pos0_ref, pos1_ref,
                bids_ref, used_ref, ff_ref, ordb_ref, uniq_ref, cex_ref):
    xf = x_ref[...]
    logits = lax.dot_general(xf, wg_ref[...], (((1,), (1,)), ((), ())),
                             preferred_element_type=jnp.float32)
    eidx = lax.broadcasted_iota(jnp.int32, (S, E), 1)
    m1 = jnp.max(logits, axis=1, keepdims=True)
    i1 = jnp.min(jnp.where(logits == m1, eidx, E), axis=1, keepdims=True)
    masked = jnp.where(eidx == i1, -jnp.inf, logits)
    m2 = jnp.max(masked, axis=1, keepdims=True)
    i2 = jnp.min(jnp.where(masked == m2, eidx, E), axis=1, keepdims=True)
    g1 = 1.0 / (1.0 + jnp.exp(m2 - m1))
    g2 = 1.0 - g1
    a1 = eidx == i1
    a2 = eidx == i2
    m = jnp.where(a1, 1.0, 0.0) + jnp.where(a2, 1.0, 0.0)

    r = lax.broadcasted_iota(jnp.int32, (CHUNK, CHUNK), 0)
    c = lax.broadcasted_iota(jnp.int32, (CHUNK, CHUNK), 1)
    ltri = jnp.where(r > c, 1.0, 0.0)
    carry = jnp.zeros((1, E), jnp.float32)
    for k in range(S // CHUNK):
        mc = m[k * CHUNK:(k + 1) * CHUNK, :]
        cex_ref[k * CHUNK:(k + 1) * CHUNK, :] = carry + lax.dot_general(
            ltri, mc, (((1,), (0,)), ((), ())),
            preferred_element_type=jnp.float32)
        carry = carry + jnp.sum(mc, axis=0, keepdims=True)

    counts = carry
    padded = jnp.ceil(counts / BLK) * BLK
    uidx_r = lax.broadcasted_iota(jnp.int32, (E, E), 0)
    uidx_c = lax.broadcasted_iota(jnp.int32, (E, E), 1)
    utri = jnp.where(uidx_r < uidx_c, 1.0, 0.0)
    start = lax.dot_general(padded, utri, (((1,), (0,)), ((), ())),
                            preferred_element_type=jnp.float32)
    base = start + cex_ref[...]
    pos0 = jnp.sum(jnp.where(a1, base, 0.0), axis=1, keepdims=True)
    pos1 = jnp.sum(jnp.where(a2, base, 0.0), axis=1, keepdims=True)
    pos0_ref[...] = pos0.astype(jnp.int32)
    pos1_ref[...] = pos1.astype(jnp.int32)
    xg0_ref[...] = g1 * xf
    xg1_ref[...] = g2 * xf

    nblk_f = padded / BLK
    cuminc = lax.dot_general(nblk_f, jnp.where(uidx_r <= uidx_c, 1.0, 0.0),
                             (((1,), (0,)), ((), ())),
                             preferred_element_type=jnp.float32)
    jj = lax.broadcasted_iota(jnp.int32, (NBLK, E), 0).astype(jnp.float32)
    bid_raw = jnp.sum(jnp.where(jj >= cuminc, 1.0, 0.0), axis=1,
                      keepdims=True)
    total = jnp.max(cuminc, axis=1, keepdims=True)
    jcol = lax.broadcasted_iota(jnp.int32, (NBLK, 1), 0).astype(jnp.float32)
    used = jnp.where(jcol < total, 1.0, 0.0)
    erow = lax.broadcasted_iota(jnp.int32, (1, E), 1).astype(jnp.float32)
    last_bid = jnp.max(jnp.where(nblk_f > 0.0, erow, 0.0), axis=1,
                       keepdims=True)
    bids = jnp.where(used > 0.0, jnp.minimum(bid_raw, float(E - 1)),
                     last_bid)
    bids_ref[...] = bids.astype(jnp.int32)
    used_ref[...] = used.astype(jnp.int32)

    ind = jnp.where(nblk_f > 0.0, 1.0, 0.0)
    rk = lax.dot_general(ind, jnp.where(uidx_r < uidx_c, 1.0, 0.0),
                         (((1,), (0,)), ((), ())),
                         preferred_element_type=jnp.float32)
    rr = lax.broadcasted_iota(jnp.int32, (E, E), 0).astype(jnp.float32)
    uq_mat = jnp.where((rk == rr) & (ind > 0.0), erow, 0.0)
    uniq = jnp.sum(uq_mat, axis=1, keepdims=True)
    nuniq = jnp.sum(ind, axis=1, keepdims=True)
    uniq_ref[...] = jnp.concatenate([uniq, nuniq], axis=0).astype(jnp.int32)
    eq = bids == erow
    ordb = jnp.sum(jnp.where(eq, rk, 0.0), axis=1, keepdims=True)
    ordb_ref[...] = ordb.astype(jnp.int32)
    prev = jnp.concatenate([-jnp.ones((1, 1), jnp.float32), bids[:-1]], axis=0)
    ff = jnp.where((used > 0.0) & (bids != prev), 1.0, 0.0)
    ff_ref[...] = ff.astype(jnp.int32)


def _route(x2, wg):
    return pl.pallas_call(
        _route_body,
        out_shape=(
            jax.ShapeDtypeStruct((S, D), jnp.float32),
            jax.ShapeDtypeStruct((S, D), jnp.float32),
            jax.ShapeDtypeStruct((S, 1), jnp.int32),
            jax.ShapeDtypeStruct((S, 1), jnp.int32),
            jax.ShapeDtypeStruct((NBLK, 1), jnp.int32),
            jax.ShapeDtypeStruct((NBLK, 1), jnp.int32),
            jax.ShapeDtypeStruct((NBLK, 1), jnp.int32),
            jax.ShapeDtypeStruct((NBLK, 1), jnp.int32),
            jax.ShapeDtypeStruct((E + 1, 1), jnp.int32),
        ),
        scratch_shapes=[pltpu.VMEM((S, E), jnp.float32)],
    )(x2, wg)


def _make_scatter():
    info = plsc.get_sparse_core_info()
    nc, ns = info.num_cores, info.num_subcores
    nw = nc * ns
    rw = S // nw
    mesh = plsc.VectorSubcoreMesh(core_axis_name="c", subcore_axis_name="s")

    @functools.partial(
        pl.kernel, mesh=mesh,
        out_type=jax.ShapeDtypeStruct((NPAD, D), jnp.float32),
        scratch_types=[
            pltpu.VMEM((rw,), jnp.int32),
            pltpu.VMEM((rw,), jnp.int32),
            pltpu.VMEM((rw, D), jnp.float32),
            pltpu.VMEM((rw, D), jnp.float32),
            pltpu.SemaphoreType.DMA,
            pltpu.SemaphoreType.DMA,
        ],
    )
    def scatter_k(xg0_hbm, xg1_hbm, pos0_hbm, pos1_hbm, xs_hbm,
                  idx0_v, idx1_v, rows0_v, rows1_v, sem0, sem1):
        wid = lax.axis_index("s") * nc + lax.axis_index("c")
        b = wid * rw
        p0 = pltpu.async_copy(pos0_hbm.at[pl.ds(b, rw)], idx0_v, sem0)
        p1 = pltpu.async_copy(pos1_hbm.at[pl.ds(b, rw)], idx1_v, sem1)
        r0 = pltpu.async_copy(xg0_hbm.at[pl.ds(b, rw)], rows0_v, sem0)
        r1 = pltpu.async_copy(xg1_hbm.at[pl.ds(b, rw)], rows1_v, sem1)
        p0.wait()
        p1.wait()
        r0.wait()
        r1.wait()
        c0 = pltpu.async_copy(rows0_v, xs_hbm.at[idx0_v], sem0)
        c1 = pltpu.async_copy(rows1_v, xs_hbm.at[idx1_v], sem1)
        c0.wait()
        c1.wait()

    return scatter_k


def _ffn_body(bids_ref, used_ref, ff_ref, ordb_ref, uniq_ref,
              xs_ref, w1_hbm, w2_hbm, ys_ref, w1b, w2b, s1, s2):
    b = pl.program_id(0)
    o = ordb_ref[b]
    nuniq = uniq_ref[E]

    @pl.when(b == 0)
    def _():
        pltpu.make_async_copy(w1_hbm.at[uniq_ref[0]], w1b.at[0],
                              s1.at[0]).start()
        pltpu.make_async_copy(w2_hbm.at[uniq_ref[0]], w2b.at[0],
                              s2.at[0]).start()

        @pl.when(nuniq > 1)
        def _():
            pltpu.make_async_copy(w1_hbm.at[uniq_ref[1]], w1b.at[1],
                                  s1.at[1]).start()
            pltpu.make_async_copy(w2_hbm.at[uniq_ref[1]], w2b.at[1],
                                  s2.at[1]).start()

        @pl.when(nuniq > 2)
        def _():
            pltpu.make_async_copy(w1_hbm.at[uniq_ref[2]], w1b.at[2],
                                  s1.at[2]).start()

    first = ff_ref[b] == 1
    o3 = lax.rem(o, 3)
    o2 = lax.rem(o, 2)

    @pl.when(first & (b > 0) & (o + 2 < nuniq))
    def _():
        nxt = uniq_ref[o + 2]
        n3 = lax.rem(o + 2, 3)

        @pl.when(n3 == 0)
        def _():
            pltpu.make_async_copy(w1_hbm.at[nxt], w1b.at[0], s1.at[0]).start()

        @pl.when(n3 == 1)
        def _():
            pltpu.make_async_copy(w1_hbm.at[nxt], w1b.at[1], s1.at[1]).start()

        @pl.when(n3 == 2)
        def _():
            pltpu.make_async_copy(w1_hbm.at[nxt], w1b.at[2], s1.at[2]).start()

    @pl.when(first & (b > 0) & (o + 1 < nuniq))
    def _():
        nx1 = uniq_ref[o + 1]
        m2 = lax.rem(o + 1, 2)

        @pl.when(m2 == 0)
        def _():
            pltpu.make_async_copy(w2_hbm.at[nx1], w2b.at[0], s2.at[0]).start()

        @pl.when(m2 == 1)
        def _():
            pltpu.make_async_copy(w2_hbm.at[nx1], w2b.at[1], s2.at[1]).start()

    @pl.when(first)
    def _():
        @pl.when(o3 == 0)
        def _():
            pltpu.make_async_copy(w1_hbm.at[0], w1b.at[0], s1.at[0]).wait()

        @pl.when(o3 == 1)
        def _():
            pltpu.make_async_copy(w1_hbm.at[0], w1b.at[1], s1.at[1]).wait()

        @pl.when(o3 == 2)
        def _():
            pltpu.make_async_copy(w1_hbm.at[0], w1b.at[2], s1.at[2]).wait()

        @pl.when(o2 == 0)
        def _():
            pltpu.make_async_copy(w2_hbm.at[0], w2b.at[0], s2.at[0]).wait()

        @pl.when(o2 == 1)
        def _():
            pltpu.make_async_copy(w2_hbm.at[0], w2b.at[1], s2.at[1]).wait()

    @pl.when(used_ref[b] == 1)
    def _():
        xb = xs_ref[...]
        w1v = w1b[o3]
        w2v = w2b[o2]
        h = lax.dot_general(xb, w1v, (((1,), (1,)), ((), ())),
                            preferred_element_type=jnp.float32)
        h = jnp.maximum(h, 0.0)
        ys_ref[...] = lax.dot_general(h, w2v, (((1,), (1,)), ((), ())),
                                      preferred_element_type=jnp.float32)


def _ffn(bids, used, ff, ordb, uniq, xs, w1, w2):
    grid_spec = pltpu.PrefetchScalarGridSpec(
        num_scalar_prefetch=5,
        grid=(NBLK,),
        in_specs=[
            pl.BlockSpec((BLK, D), lambda b, *_: (b, 0)),
            pl.BlockSpec(memory_space=pl.ANY),
            pl.BlockSpec(memory_space=pl.ANY),
        ],
        out_specs=pl.BlockSpec((BLK, D), lambda b, *_: (b, 0)),
        scratch_shapes=[
            pltpu.VMEM((3, DFF, D), jnp.float32),
            pltpu.VMEM((2, D, DFF), jnp.float32),
            pltpu.SemaphoreType.DMA((3,)),
            pltpu.SemaphoreType.DMA((2,)),
        ],
    )
    return pl.pallas_call(
        _ffn_body,
        grid_spec=grid_spec,
        out_shape=jax.ShapeDtypeStruct((NPAD, D), jnp.float32),
        compiler_params=pltpu.CompilerParams(
            dimension_semantics=("arbitrary",)),
    )(bids, used, ff, ordb, uniq, xs, w1, w2)


def _make_combine():
    info = plsc.get_sparse_core_info()
    nc, ns = info.num_cores, info.num_subcores
    nw = nc * ns
    rw = S // nw
    nv = D // 16
    mesh = plsc.VectorSubcoreMesh(core_axis_name="c", subcore_axis_name="s")

    @functools.partial(
        pl.kernel, mesh=mesh,
        out_type=jax.ShapeDtypeStruct((S, D), jnp.float32),
        scratch_types=[
            pltpu.VMEM((rw,), jnp.int32),
            pltpu.VMEM((rw,), jnp.int32),
            pltpu.VMEM((rw, D), jnp.float32),
            pltpu.VMEM((rw, D), jnp.float32),
            pltpu.SemaphoreType.DMA,
            pltpu.SemaphoreType.DMA,
        ],
    )
    def combine_k(ys_hbm, pos0_hbm, pos1_hbm, out_hbm,
                  idx0_v, idx1_v, bufa_v, bufb_v, sem0, sem1):
        wid = lax.axis_index("s") * nc + lax.axis_index("c")
        b = wid * rw
        p0 = pltpu.async_copy(pos0_hbm.at[pl.ds(b, rw)], idx0_v, sem0)
        p1 = pltpu.async_copy(pos1_hbm.at[pl.ds(b, rw)], idx1_v, sem1)
        p0.wait()
        p1.wait()
        ca = pltpu.async_copy(ys_hbm.at[idx0_v], bufa_v, sem0)
        cb = pltpu.async_copy(ys_hbm.at[idx1_v], bufb_v, sem1)
        ca.wait()
        cb.wait()

        def row(r, _):
            for cc in range(nv):
                sl = pl.ds(cc * 16, 16)
                bufa_v[r, sl] = bufa_v[r, sl] + bufb_v[r, sl]
            return 0

        lax.fori_loop(0, rw, row, 0)
        pltpu.sync_copy(bufa_v, out_hbm.at[pl.ds(b, rw)])

    return combine_k


def kernel(x, Wg, W1, W2):
    bsz, s, d = x.shape
    x2 = x.reshape(s, d)

    xg0, xg1, pos0, pos1, bids, used, ff, ordb, uniq = _route(x2, Wg)
    pos0 = pos0.reshape(S)
    pos1 = pos1.reshape(S)
    bids = bids.reshape(NBLK)
    used = used.reshape(NBLK)
    ff = ff.reshape(NBLK)
    ordb = ordb.reshape(NBLK)
    uniq = uniq.reshape(E + 1)

    xs = _make_scatter()(xg0, xg1, pos0, pos1)
    ys = _ffn(bids, used, ff, ordb, uniq, xs, W1, W2)
    out = _make_combine()(ys, pos0, pos1)
    return out.reshape(bsz, s, d)
